# Initial kernel scaffold; baseline (speedup 1.0000x reference)
import jax
import jax.numpy as jnp
from jax.experimental import pallas as pl


def kernel(x, edge_index, W_pre, b_pre, W1, b1, W2, b2, W_post, b_post):
    raise NotImplementedError("write your pallas kernel here")


# speedup vs baseline: 19.9576x; 19.9576x over previous
import functools

import jax
import jax.numpy as jnp
from jax import lax
from jax.experimental import pallas as pl
from jax.experimental.pallas import tpu as pltpu
from jax.experimental.pallas import tpu_sc as plsc

N = 10000
E = 320000
H = 128
C = 40
NC, NS = 2, 16
K = 125
EB = E // K
RPT = EB // (NC * NS)
NPAD = 10240
NPT = NPAD // NS
BLK = 2000
_F32 = jnp.float32


_MESH = plsc.VectorSubcoreMesh(
    core_axis_name="c", subcore_axis_name="s", num_cores=NC, num_subcores=NS)


def _sc_count_body(dst_hbm, ones_hbm, zeros_hbm, cnt_out, didx_v, ones_v, cacc):
    c = lax.axis_index("c")
    s = lax.axis_index("s")
    wid = c * NS + s
    pltpu.sync_copy(zeros_hbm, cacc.at[pl.ds(s * NPT, NPT)])
    pltpu.sync_copy(dst_hbm.at[pl.ds(wid * RPT, RPT)], didx_v)
    pltpu.sync_copy(ones_hbm, ones_v)
    plsc.subcore_barrier()

    def body(j, carry):
        pltpu.sync_copy(ones_v, cacc.at[didx_v.at[j]], add=True)
        return carry

    lax.fori_loop(0, RPT, body, 0)
    plsc.subcore_barrier()
    pltpu.sync_copy(cacc.at[pl.ds(s * NPT, NPT)],
                    cnt_out.at[c].at[pl.ds(s * NPT, NPT)])


@jax.jit
def _sc_count(dst2d, ones_c, zeros_c):
    return pl.kernel(
        _sc_count_body,
        out_type=jax.ShapeDtypeStruct((NC, NPAD, H), _F32),
        mesh=_MESH,
        scratch_types=[
            pltpu.VMEM((RPT, K), jnp.int32),
            pltpu.VMEM((K, H), _F32),
            pltpu.VMEM_SHARED((NPAD, H), _F32),
        ],
    )(dst2d, ones_c, zeros_c)


def _sc_scatter_body(tab_hbm, src_hbm, dst_hbm, zeros_hbm, acc_out,
                     sidx_v, didx_v, rows_v, sem, sacc):
    c = lax.axis_index("c")
    s = lax.axis_index("s")
    wid = c * NS + s
    pltpu.sync_copy(zeros_hbm, sacc.at[pl.ds(s * NPT, NPT)])
    base = wid * RPT
    pltpu.sync_copy(src_hbm.at[pl.ds(base, RPT)], sidx_v)
    pltpu.sync_copy(dst_hbm.at[pl.ds(base, RPT)], didx_v)
    plsc.subcore_barrier()

    def body(j, carry):
        pltpu.async_copy(tab_hbm.at[sidx_v.at[j]], rows_v, sem).wait()
        pltpu.sync_copy(rows_v, sacc.at[didx_v.at[j]], add=True)
        return carry

    lax.fori_loop(0, RPT, body, 0)
    plsc.subcore_barrier()
    pltpu.sync_copy(sacc.at[pl.ds(s * NPT, NPT)],
                    acc_out.at[c].at[pl.ds(s * NPT, NPT)])


@jax.jit
def _sc_scatter(tab, src2d, dst2d, zeros_a):
    return pl.kernel(
        _sc_scatter_body,
        out_type=jax.ShapeDtypeStruct((NC, NPAD, H), _F32),
        mesh=_MESH,
        scratch_types=[
            pltpu.VMEM((RPT, K), jnp.int32),
            pltpu.VMEM((RPT, K), jnp.int32),
            pltpu.VMEM((K, H), _F32),
            pltpu.SemaphoreType.DMA,
            pltpu.VMEM_SHARED((NPAD, H), _F32),
        ],
    )(tab, src2d, dst2d, zeros_a)


def _dinv_of(cnt_blk):
    deg = cnt_blk[0, :, 0:1] + cnt_blk[1, :, 0:1] + 1.0
    return lax.rsqrt(deg)


DW = 8


def _mm2_body(x_ref, wp_ref, bp_ref, w1_ref, o_ref):
    h = jnp.dot(x_ref[...], wp_ref[...], preferred_element_type=_F32)
    h = h + bp_ref[...]
    o_ref[...] = jnp.dot(h, w1_ref[...], preferred_element_type=_F32)


@jax.jit
def _tc_mm2(x, W_pre, b_pre, W1):
    return pl.pallas_call(
        _mm2_body,
        grid=(N // BLK,),
        in_specs=[
            pl.BlockSpec((BLK, H), lambda i: (i, 0)),
            pl.BlockSpec((H, H), lambda i: (0, 0)),
            pl.BlockSpec((1, H), lambda i: (0, 0)),
            pl.BlockSpec((H, H), lambda i: (0, 0)),
        ],
        out_specs=pl.BlockSpec((BLK, H), lambda i: (i, 0)),
        out_shape=jax.ShapeDtypeStruct((N, H), _F32),
    )(x, W_pre, b_pre, W1)


def _scale_body(xw_ref, cnt_ref, tab_ref, dinv_ref):
    dinv = _dinv_of(cnt_ref)
    tab_ref[...] = dinv * xw_ref[...]
    dinv_ref[...] = jnp.broadcast_to(dinv, (BLK, DW))


@jax.jit
def _tc_scale(xw, cnt):
    return pl.pallas_call(
        _scale_body,
        grid=(N // BLK,),
        in_specs=[
            pl.BlockSpec((BLK, H), lambda i: (i, 0)),
            pl.BlockSpec((NC, BLK, H), lambda i: (0, i, 0)),
        ],
        out_specs=[
            pl.BlockSpec((BLK, H), lambda i: (i, 0)),
            pl.BlockSpec((BLK, DW), lambda i: (i, 0)),
        ],
        out_shape=[
            jax.ShapeDtypeStruct((N, H), _F32),
            jax.ShapeDtypeStruct((N, DW), _F32),
        ],
    )(xw, cnt)


def _layer_body(acc_ref, tab_ref, dinv_ref, w_ref, b_ref, out_ref):
    dinv = dinv_ref[:, 0:1]
    agg = acc_ref[0] + acc_ref[1] + tab_ref[...]
    h = jnp.maximum(dinv * agg + b_ref[...], 0.0)
    xw = jnp.dot(h, w_ref[...], preferred_element_type=_F32)
    out_ref[...] = dinv * xw


@jax.jit
def _tc_layer(acc, tab, dinv8, W, b):
    return pl.pallas_call(
        _layer_body,
        grid=(N // BLK,),
        in_specs=[
            pl.BlockSpec((NC, BLK, H), lambda i: (0, i, 0)),
            pl.BlockSpec((BLK, H), lambda i: (i, 0)),
            pl.BlockSpec((BLK, DW), lambda i: (i, 0)),
            pl.BlockSpec((H, H), lambda i: (0, 0)),
            pl.BlockSpec((1, H), lambda i: (0, 0)),
        ],
        out_specs=pl.BlockSpec((BLK, H), lambda i: (i, 0)),
        out_shape=jax.ShapeDtypeStruct((N, H), _F32),
    )(acc, tab, dinv8, W, b)


def _final_body(acc_ref, tab_ref, dinv_ref, b2_ref, wpost_ref, bpost_ref, out_ref):
    dinv = dinv_ref[:, 0:1]
    agg = acc_ref[0] + acc_ref[1] + tab_ref[...]
    h = jnp.maximum(dinv * agg + b2_ref[...], 0.0)
    out_ref[...] = jnp.dot(h, wpost_ref[...],
                           preferred_element_type=_F32) + bpost_ref[...]


@jax.jit
def _tc_final(acc, tab, dinv8, b2, W_post, b_post):
    return pl.pallas_call(
        _final_body,
        grid=(N // BLK,),
        in_specs=[
            pl.BlockSpec((NC, BLK, H), lambda i: (0, i, 0)),
            pl.BlockSpec((BLK, H), lambda i: (i, 0)),
            pl.BlockSpec((BLK, DW), lambda i: (i, 0)),
            pl.BlockSpec((1, H), lambda i: (0, 0)),
            pl.BlockSpec((H, C), lambda i: (0, 0)),
            pl.BlockSpec((1, C), lambda i: (0, 0)),
        ],
        out_specs=pl.BlockSpec((BLK, C), lambda i: (i, 0)),
        out_shape=jax.ShapeDtypeStruct((N, C), _F32),
    )(acc, tab, dinv8, b2, W_post, b_post)


def kernel(x, edge_index, W_pre, b_pre, W1, b1, W2, b2, W_post, b_post):
    src2d = edge_index[0].reshape(EB, K)
    dst2d = edge_index[1].reshape(EB, K)
    ones_c = jnp.ones((K, H), _F32)
    zeros_a = jnp.zeros((NPT, H), _F32)

    xw1 = _tc_mm2(x, W_pre, b_pre.reshape(1, H), W1)
    cnt = _sc_count(dst2d, ones_c, zeros_a)
    tab1, dinv8 = _tc_scale(xw1, cnt)
    acc1 = _sc_scatter(tab1, src2d, dst2d, zeros_a)
    tab2 = _tc_layer(acc1, tab1, dinv8, W2, b1.reshape(1, H))
    acc2 = _sc_scatter(tab2, src2d, dst2d, zeros_a)
    return _tc_final(acc2, tab2, dinv8, b2.reshape(1, H),
                     W_post, b_post.reshape(1, C))


# speedup vs baseline: 24.3033x; 1.2177x over previous
import functools

import jax
import jax.numpy as jnp
from jax import lax
from jax.experimental import pallas as pl
from jax.experimental.pallas import tpu as pltpu
from jax.experimental.pallas import tpu_sc as plsc

N = 10000
E = 320000
H = 128
C = 40
NC, NS = 2, 16
K = 125
EB = E // K
RPT = EB // (NC * NS)
NPAD = 10240
NPT = NPAD // NS
NPH = 2
PH = RPT // NPH
BLK = 2000
_F32 = jnp.float32


_MESH = plsc.VectorSubcoreMesh(
    core_axis_name="c", subcore_axis_name="s", num_cores=NC, num_subcores=NS)


def _sc_count_body(dst_hbm, ones_hbm, zeros_hbm, cnt_out, didx_v, ones_v, cacc):
    c = lax.axis_index("c")
    s = lax.axis_index("s")
    wid = c * NS + s
    pltpu.sync_copy(zeros_hbm, cacc.at[pl.ds(s * NPT, NPT)])
    pltpu.sync_copy(dst_hbm.at[pl.ds(wid * RPT, RPT)], didx_v)
    pltpu.sync_copy(ones_hbm, ones_v)
    plsc.subcore_barrier()

    def body(j, carry):
        pltpu.sync_copy(ones_v, cacc.at[didx_v.at[j]], add=True)
        return carry

    lax.fori_loop(0, RPT, body, 0)
    plsc.subcore_barrier()
    pltpu.sync_copy(cacc.at[pl.ds(s * NPT, NPT)],
                    cnt_out.at[c].at[pl.ds(s * NPT, NPT)])


@jax.jit
def _sc_count(dst2d, ones_c, zeros_c):
    return pl.kernel(
        _sc_count_body,
        out_type=jax.ShapeDtypeStruct((NC, NPAD, H), _F32),
        mesh=_MESH,
        scratch_types=[
            pltpu.VMEM((RPT, K), jnp.int32),
            pltpu.VMEM((K, H), _F32),
            pltpu.VMEM_SHARED((NPAD, H), _F32),
        ],
    )(dst2d, ones_c, zeros_c)


def _sc_scatter_body(tab_hbm, src_hbm, dst_hbm, zeros_hbm, acc_out,
                     sidx_v, didx_v, rows0_v, rows1_v, sem0, sem1, sacc):
    c = lax.axis_index("c")
    s = lax.axis_index("s")
    wid = c * NS + s
    pltpu.sync_copy(zeros_hbm, sacc.at[pl.ds(s * NPT, NPT)])
    base = wid * RPT
    plsc.subcore_barrier()

    for q in range(NPH):
        pltpu.sync_copy(src_hbm.at[pl.ds(base + q * PH, PH)], sidx_v)
        pltpu.sync_copy(dst_hbm.at[pl.ds(base + q * PH, PH)], didx_v)
        pltpu.async_copy(tab_hbm.at[sidx_v.at[0]], rows0_v, sem0)

        def body(p, carry):
            j0 = p * 2
            pltpu.make_async_copy(
                tab_hbm.at[sidx_v.at[j0]], rows0_v, sem0).wait()
            pltpu.async_copy(tab_hbm.at[sidx_v.at[j0 + 1]], rows1_v, sem1)
            pltpu.sync_copy(rows0_v, sacc.at[didx_v.at[j0]], add=True)
            pltpu.make_async_copy(
                tab_hbm.at[sidx_v.at[j0 + 1]], rows1_v, sem1).wait()

            @pl.when(p + 1 < PH // 2)
            def _():
                pltpu.async_copy(tab_hbm.at[sidx_v.at[j0 + 2]], rows0_v, sem0)

            pltpu.sync_copy(rows1_v, sacc.at[didx_v.at[j0 + 1]], add=True)
            return carry

        lax.fori_loop(0, PH // 2, body, 0)
    plsc.subcore_barrier()
    pltpu.sync_copy(sacc.at[pl.ds(s * NPT, NPT)],
                    acc_out.at[c].at[pl.ds(s * NPT, NPT)])


@jax.jit
def _sc_scatter(tab, src2d, dst2d, zeros_a):
    return pl.kernel(
        _sc_scatter_body,
        out_type=jax.ShapeDtypeStruct((NC, NPAD, H), _F32),
        mesh=_MESH,
        scratch_types=[
            pltpu.VMEM((PH, K), jnp.int32),
            pltpu.VMEM((PH, K), jnp.int32),
            pltpu.VMEM((K, H), _F32),
            pltpu.VMEM((K, H), _F32),
            pltpu.SemaphoreType.DMA,
            pltpu.SemaphoreType.DMA,
            pltpu.VMEM_SHARED((NPAD, H), _F32),
        ],
    )(tab, src2d, dst2d, zeros_a)


def _dinv_of(cnt_blk):
    deg = cnt_blk[0, :, 0:1] + cnt_blk[1, :, 0:1] + 1.0
    return lax.rsqrt(deg)


DW = 8


def _mm2_body(x_ref, wp_ref, bp_ref, w1_ref, o_ref):
    h = jnp.dot(x_ref[...], wp_ref[...], preferred_element_type=_F32)
    h = h + bp_ref[...]
    o_ref[...] = jnp.dot(h, w1_ref[...], preferred_element_type=_F32)


@jax.jit
def _tc_mm2(x, W_pre, b_pre, W1):
    return pl.pallas_call(
        _mm2_body,
        grid=(N // BLK,),
        in_specs=[
            pl.BlockSpec((BLK, H), lambda i: (i, 0)),
            pl.BlockSpec((H, H), lambda i: (0, 0)),
            pl.BlockSpec((1, H), lambda i: (0, 0)),
            pl.BlockSpec((H, H), lambda i: (0, 0)),
        ],
        out_specs=pl.BlockSpec((BLK, H), lambda i: (i, 0)),
        out_shape=jax.ShapeDtypeStruct((N, H), _F32),
    )(x, W_pre, b_pre, W1)


def _scale_body(xw_ref, cnt_ref, tab_ref, dinv_ref):
    dinv = _dinv_of(cnt_ref)
    tab_ref[...] = dinv * xw_ref[...]
    dinv_ref[...] = jnp.broadcast_to(dinv, (BLK, DW))


@jax.jit
def _tc_scale(xw, cnt):
    return pl.pallas_call(
        _scale_body,
        grid=(N // BLK,),
        in_specs=[
            pl.BlockSpec((BLK, H), lambda i: (i, 0)),
            pl.BlockSpec((NC, BLK, H), lambda i: (0, i, 0)),
        ],
        out_specs=[
            pl.BlockSpec((BLK, H), lambda i: (i, 0)),
            pl.BlockSpec((BLK, DW), lambda i: (i, 0)),
        ],
        out_shape=[
            jax.ShapeDtypeStruct((N, H), _F32),
            jax.ShapeDtypeStruct((N, DW), _F32),
        ],
    )(xw, cnt)


def _layer_body(acc_ref, tab_ref, dinv_ref, w_ref, b_ref, out_ref):
    dinv = dinv_ref[:, 0:1]
    agg = acc_ref[0] + acc_ref[1] + tab_ref[...]
    h = jnp.maximum(dinv * agg + b_ref[...], 0.0)
    xw = jnp.dot(h, w_ref[...], preferred_element_type=_F32)
    out_ref[...] = dinv * xw


@jax.jit
def _tc_layer(acc, tab, dinv8, W, b):
    return pl.pallas_call(
        _layer_body,
        grid=(N // BLK,),
        in_specs=[
            pl.BlockSpec((NC, BLK, H), lambda i: (0, i, 0)),
            pl.BlockSpec((BLK, H), lambda i: (i, 0)),
            pl.BlockSpec((BLK, DW), lambda i: (i, 0)),
            pl.BlockSpec((H, H), lambda i: (0, 0)),
            pl.BlockSpec((1, H), lambda i: (0, 0)),
        ],
        out_specs=pl.BlockSpec((BLK, H), lambda i: (i, 0)),
        out_shape=jax.ShapeDtypeStruct((N, H), _F32),
    )(acc, tab, dinv8, W, b)


def _final_body(acc_ref, tab_ref, dinv_ref, b2_ref, wpost_ref, bpost_ref, out_ref):
    dinv = dinv_ref[:, 0:1]
    agg = acc_ref[0] + acc_ref[1] + tab_ref[...]
    h = jnp.maximum(dinv * agg + b2_ref[...], 0.0)
    out_ref[...] = jnp.dot(h, wpost_ref[...],
                           preferred_element_type=_F32) + bpost_ref[...]


@jax.jit
def _tc_final(acc, tab, dinv8, b2, W_post, b_post):
    return pl.pallas_call(
        _final_body,
        grid=(N // BLK,),
        in_specs=[
            pl.BlockSpec((NC, BLK, H), lambda i: (0, i, 0)),
            pl.BlockSpec((BLK, H), lambda i: (i, 0)),
            pl.BlockSpec((BLK, DW), lambda i: (i, 0)),
            pl.BlockSpec((1, H), lambda i: (0, 0)),
            pl.BlockSpec((H, C), lambda i: (0, 0)),
            pl.BlockSpec((1, C), lambda i: (0, 0)),
        ],
        out_specs=pl.BlockSpec((BLK, C), lambda i: (i, 0)),
        out_shape=jax.ShapeDtypeStruct((N, C), _F32),
    )(acc, tab, dinv8, b2, W_post, b_post)


def kernel(x, edge_index, W_pre, b_pre, W1, b1, W2, b2, W_post, b_post):
    src2d = edge_index[0].reshape(EB, K)
    dst2d = edge_index[1].reshape(EB, K)
    ones_c = jnp.ones((K, H), _F32)
    zeros_a = jnp.zeros((NPT, H), _F32)

    xw1 = _tc_mm2(x, W_pre, b_pre.reshape(1, H), W1)
    cnt = _sc_count(dst2d, ones_c, zeros_a)
    tab1, dinv8 = _tc_scale(xw1, cnt)
    acc1 = _sc_scatter(tab1, src2d, dst2d, zeros_a)
    tab2 = _tc_layer(acc1, tab1, dinv8, W2, b1.reshape(1, H))
    acc2 = _sc_scatter(tab2, src2d, dst2d, zeros_a)
    return _tc_final(acc2, tab2, dinv8, b2.reshape(1, H),
                     W_post, b_post.reshape(1, C))


# speedup vs baseline: 28.1911x; 1.1600x over previous
import functools

import jax
import jax.numpy as jnp
from jax import lax
from jax.experimental import pallas as pl
from jax.experimental.pallas import tpu as pltpu
from jax.experimental.pallas import tpu_sc as plsc

N = 10000
E = 320000
H = 128
C = 40
NC, NS = 2, 16
K = 125
EB = E // K
RPT = EB // (NC * NS)
NPAD = 10240
NPT = NPAD // NS
NPH = 2
PH = RPT // NPH
BLK = 2000
_F32 = jnp.float32


_MESH = plsc.VectorSubcoreMesh(
    core_axis_name="c", subcore_axis_name="s", num_cores=NC, num_subcores=NS)


def _sc_count_body(dst_hbm, ones_hbm, zeros_hbm, cnt_out, didx_v, ones_v, cacc):
    c = lax.axis_index("c")
    s = lax.axis_index("s")
    wid = c * NS + s
    pltpu.sync_copy(zeros_hbm, cacc.at[pl.ds(s * NPT, NPT)])
    pltpu.sync_copy(dst_hbm.at[pl.ds(wid * RPT, RPT)], didx_v)
    pltpu.sync_copy(ones_hbm, ones_v)
    plsc.subcore_barrier()

    def body(j, carry):
        pltpu.sync_copy(ones_v, cacc.at[didx_v.at[j]], add=True)
        return carry

    lax.fori_loop(0, RPT, body, 0)
    plsc.subcore_barrier()
    pltpu.sync_copy(cacc.at[pl.ds(s * NPT, NPT)],
                    cnt_out.at[c].at[pl.ds(s * NPT, NPT)])


@jax.jit
def _sc_count(dst2d, ones_c, zeros_c):
    return pl.kernel(
        _sc_count_body,
        out_type=jax.ShapeDtypeStruct((NC, NPAD), _F32),
        mesh=_MESH,
        scratch_types=[
            pltpu.VMEM((RPT, K), jnp.int32),
            pltpu.VMEM((K,), _F32),
            pltpu.VMEM_SHARED((NPAD,), _F32),
        ],
    )(dst2d, ones_c, zeros_c)


def _sc_scatter_body(tab_hbm, src_hbm, dst_hbm, zeros_hbm, acc_out,
                     sidx_v, didx_v, rows0_v, rows1_v, sem0, sem1, sacc):
    c = lax.axis_index("c")
    s = lax.axis_index("s")
    wid = c * NS + s
    pltpu.sync_copy(zeros_hbm, sacc.at[pl.ds(s * NPT, NPT)])
    base = wid * RPT
    plsc.subcore_barrier()

    for q in range(NPH):
        pltpu.sync_copy(src_hbm.at[pl.ds(base + q * PH, PH)], sidx_v)
        pltpu.sync_copy(dst_hbm.at[pl.ds(base + q * PH, PH)], didx_v)
        pltpu.async_copy(tab_hbm.at[sidx_v.at[0]], rows0_v, sem0)

        def body(p, carry):
            j0 = p * 2
            pltpu.make_async_copy(
                tab_hbm.at[sidx_v.at[j0]], rows0_v, sem0).wait()
            pltpu.async_copy(tab_hbm.at[sidx_v.at[j0 + 1]], rows1_v, sem1)
            pltpu.sync_copy(rows0_v, sacc.at[didx_v.at[j0]], add=True)
            pltpu.make_async_copy(
                tab_hbm.at[sidx_v.at[j0 + 1]], rows1_v, sem1).wait()

            @pl.when(p + 1 < PH // 2)
            def _():
                pltpu.async_copy(tab_hbm.at[sidx_v.at[j0 + 2]], rows0_v, sem0)

            pltpu.sync_copy(rows1_v, sacc.at[didx_v.at[j0 + 1]], add=True)
            return carry

        lax.fori_loop(0, PH // 2, body, 0)
    plsc.subcore_barrier()
    pltpu.sync_copy(sacc.at[pl.ds(s * NPT, NPT)],
                    acc_out.at[c].at[pl.ds(s * NPT, NPT)])


@jax.jit
def _sc_scatter(tab, src2d, dst2d, zeros_a):
    return pl.kernel(
        _sc_scatter_body,
        out_type=jax.ShapeDtypeStruct((NC, NPAD, H), _F32),
        mesh=_MESH,
        scratch_types=[
            pltpu.VMEM((PH, K), jnp.int32),
            pltpu.VMEM((PH, K), jnp.int32),
            pltpu.VMEM((K, H), _F32),
            pltpu.VMEM((K, H), _F32),
            pltpu.SemaphoreType.DMA,
            pltpu.SemaphoreType.DMA,
            pltpu.VMEM_SHARED((NPAD, H), _F32),
        ],
    )(tab, src2d, dst2d, zeros_a)


def _dinv_of(cnt_blk):
    deg = cnt_blk[0, :, 0:1] + cnt_blk[1, :, 0:1] + 1.0
    return lax.rsqrt(deg)


DW = 8


def _mm2_body(x_ref, wp_ref, bp_ref, w1_ref, o_ref):
    h = jnp.dot(x_ref[...], wp_ref[...], preferred_element_type=_F32)
    h = h + bp_ref[...]
    o_ref[...] = jnp.dot(h, w1_ref[...], preferred_element_type=_F32)


@jax.jit
def _tc_mm2(x, W_pre, b_pre, W1):
    return pl.pallas_call(
        _mm2_body,
        grid=(N // BLK,),
        in_specs=[
            pl.BlockSpec((BLK, H), lambda i: (i, 0)),
            pl.BlockSpec((H, H), lambda i: (0, 0)),
            pl.BlockSpec((1, H), lambda i: (0, 0)),
            pl.BlockSpec((H, H), lambda i: (0, 0)),
        ],
        out_specs=pl.BlockSpec((BLK, H), lambda i: (i, 0)),
        out_shape=jax.ShapeDtypeStruct((N, H), _F32),
    )(x, W_pre, b_pre, W1)


def _scale_body(xw_ref, cnt_ref, tab_ref, dinv_ref):
    deg = cnt_ref[:, 0:1] + cnt_ref[:, 1:2] + 1.0
    dinv = lax.rsqrt(deg)
    tab_ref[...] = dinv * xw_ref[...]
    dinv_ref[...] = jnp.broadcast_to(dinv, (BLK, DW))


@jax.jit
def _tc_scale(xw, cnt):
    return pl.pallas_call(
        _scale_body,
        grid=(N // BLK,),
        in_specs=[
            pl.BlockSpec((BLK, H), lambda i: (i, 0)),
            pl.BlockSpec((BLK, NC), lambda i: (i, 0)),
        ],
        out_specs=[
            pl.BlockSpec((BLK, H), lambda i: (i, 0)),
            pl.BlockSpec((BLK, DW), lambda i: (i, 0)),
        ],
        out_shape=[
            jax.ShapeDtypeStruct((N, H), _F32),
            jax.ShapeDtypeStruct((N, DW), _F32),
        ],
    )(xw, cnt)


def _layer_body(acc_ref, tab_ref, dinv_ref, w_ref, b_ref, out_ref):
    dinv = dinv_ref[:, 0:1]
    agg = acc_ref[0] + acc_ref[1] + tab_ref[...]
    h = jnp.maximum(dinv * agg + b_ref[...], 0.0)
    xw = jnp.dot(h, w_ref[...], preferred_element_type=_F32)
    out_ref[...] = dinv * xw


@jax.jit
def _tc_layer(acc, tab, dinv8, W, b):
    return pl.pallas_call(
        _layer_body,
        grid=(N // BLK,),
        in_specs=[
            pl.BlockSpec((NC, BLK, H), lambda i: (0, i, 0)),
            pl.BlockSpec((BLK, H), lambda i: (i, 0)),
            pl.BlockSpec((BLK, DW), lambda i: (i, 0)),
            pl.BlockSpec((H, H), lambda i: (0, 0)),
            pl.BlockSpec((1, H), lambda i: (0, 0)),
        ],
        out_specs=pl.BlockSpec((BLK, H), lambda i: (i, 0)),
        out_shape=jax.ShapeDtypeStruct((N, H), _F32),
    )(acc, tab, dinv8, W, b)


def _final_body(acc_ref, tab_ref, dinv_ref, b2_ref, wpost_ref, bpost_ref, out_ref):
    dinv = dinv_ref[:, 0:1]
    agg = acc_ref[0] + acc_ref[1] + tab_ref[...]
    h = jnp.maximum(dinv * agg + b2_ref[...], 0.0)
    out_ref[...] = jnp.dot(h, wpost_ref[...],
                           preferred_element_type=_F32) + bpost_ref[...]


@jax.jit
def _tc_final(acc, tab, dinv8, b2, W_post, b_post):
    return pl.pallas_call(
        _final_body,
        grid=(N // BLK,),
        in_specs=[
            pl.BlockSpec((NC, BLK, H), lambda i: (0, i, 0)),
            pl.BlockSpec((BLK, H), lambda i: (i, 0)),
            pl.BlockSpec((BLK, DW), lambda i: (i, 0)),
            pl.BlockSpec((1, H), lambda i: (0, 0)),
            pl.BlockSpec((H, C), lambda i: (0, 0)),
            pl.BlockSpec((1, C), lambda i: (0, 0)),
        ],
        out_specs=pl.BlockSpec((BLK, C), lambda i: (i, 0)),
        out_shape=jax.ShapeDtypeStruct((N, C), _F32),
    )(acc, tab, dinv8, b2, W_post, b_post)


def kernel(x, edge_index, W_pre, b_pre, W1, b1, W2, b2, W_post, b_post):
    src2d = edge_index[0].reshape(EB, K)
    dst2d = edge_index[1].reshape(EB, K)
    ones_c = jnp.ones((K,), _F32)
    zeros_c = jnp.zeros((NPT,), _F32)
    zeros_a = jnp.zeros((NPT, H), _F32)

    xw1 = _tc_mm2(x, W_pre, b_pre.reshape(1, H), W1)
    cnt = _sc_count(dst2d, ones_c, zeros_c)
    tab1, dinv8 = _tc_scale(xw1, cnt.T)
    acc1 = _sc_scatter(tab1, src2d, dst2d, zeros_a)
    tab2 = _tc_layer(acc1, tab1, dinv8, W2, b1.reshape(1, H))
    acc2 = _sc_scatter(tab2, src2d, dst2d, zeros_a)
    return _tc_final(acc2, tab2, dinv8, b2.reshape(1, H),
                     W_post, b_post.reshape(1, C))


# speedup vs baseline: 29.1139x; 1.0327x over previous
import jax
import jax.numpy as jnp
from jax import lax
from jax.experimental import pallas as pl
from jax.experimental.pallas import tpu as pltpu
from jax.experimental.pallas import tpu_sc as plsc

N = 10000
E = 320000
H = 128
C = 40
NC, NS = 2, 16
K = 125
EB = E // K
RPT = EB // (NC * NS)
NPAD = 10240
NPT = NPAD // NS
NPH = 2
PH = RPT // NPH
KR = 128
BLK = 2000
_F32 = jnp.float32


_MESH = plsc.VectorSubcoreMesh(
    core_axis_name="c", subcore_axis_name="s", num_cores=NC, num_subcores=NS)


def _sc_count_body(dst_hbm, ones_hbm, zeros_hbm, cnt_out, didx_v, ones_v, cacc):
    c = lax.axis_index("c")
    s = lax.axis_index("s")
    wid = c * NS + s
    pltpu.sync_copy(zeros_hbm, cacc.at[pl.ds(s * NPT, NPT)])
    pltpu.sync_copy(dst_hbm.at[pl.ds(wid * RPT, RPT)], didx_v)
    pltpu.sync_copy(ones_hbm, ones_v)
    plsc.subcore_barrier()

    def body(j, carry):
        pltpu.sync_copy(ones_v, cacc.at[didx_v.at[j]], add=True)
        return carry

    lax.fori_loop(0, RPT, body, 0)
    plsc.subcore_barrier()
    pltpu.sync_copy(cacc.at[pl.ds(s * NPT, NPT)],
                    cnt_out.at[c].at[pl.ds(s * NPT, NPT)])


@jax.jit
def _sc_count(dst2d, ones_c, zeros_c):
    return pl.kernel(
        _sc_count_body,
        out_type=jax.ShapeDtypeStruct((NC, NPAD), _F32),
        mesh=_MESH,
        scratch_types=[
            pltpu.VMEM((RPT, K), jnp.int32),
            pltpu.VMEM((K,), _F32),
            pltpu.VMEM_SHARED((NPAD,), _F32),
        ],
    )(dst2d, ones_c, zeros_c)


def _sc_scatter_body(tab_hbm, src_hbm, dst_hbm, acc_out,
                     sidx_v, didx_v, rows0_v, rows1_v, sem0, sem1, sacc):
    c = lax.axis_index("c")
    s = lax.axis_index("s")
    wid = c * NS + s

    base = wid * RPT
    pltpu.sync_copy(src_hbm.at[pl.ds(base, PH)], sidx_v)
    pltpu.sync_copy(dst_hbm.at[pl.ds(base, PH)], didx_v)
    pltpu.async_copy(tab_hbm.at[sidx_v.at[0]], rows0_v.at[pl.ds(0, K)], sem0)

    zvec = jnp.zeros((16,), _F32)

    def zbody(i, carry):
        for l in range(8):
            rows1_v[i, pl.ds(l * 16, 16)] = zvec
        return carry

    lax.fori_loop(0, KR, zbody, 0)
    for r in range(NPT // KR):
        pltpu.sync_copy(rows1_v, sacc.at[pl.ds(s * NPT + r * KR, KR)])
    plsc.subcore_barrier()

    for q in range(NPH):
        if q > 0:
            pltpu.sync_copy(src_hbm.at[pl.ds(base + q * PH, PH)], sidx_v)
            pltpu.sync_copy(dst_hbm.at[pl.ds(base + q * PH, PH)], didx_v)
            pltpu.async_copy(
                tab_hbm.at[sidx_v.at[0]], rows0_v.at[pl.ds(0, K)], sem0)

        def body(p, carry):
            j0 = p * 2
            pltpu.make_async_copy(
                tab_hbm.at[sidx_v.at[j0]], rows0_v.at[pl.ds(0, K)],
                sem0).wait()
            pltpu.async_copy(
                tab_hbm.at[sidx_v.at[j0 + 1]], rows1_v.at[pl.ds(0, K)], sem1)
            pltpu.sync_copy(
                rows0_v.at[pl.ds(0, K)], sacc.at[didx_v.at[j0]], add=True)
            pltpu.make_async_copy(
                tab_hbm.at[sidx_v.at[j0 + 1]], rows1_v.at[pl.ds(0, K)],
                sem1).wait()

            @pl.when(p + 1 < PH // 2)
            def _():
                pltpu.async_copy(
                    tab_hbm.at[sidx_v.at[j0 + 2]], rows0_v.at[pl.ds(0, K)],
                    sem0)

            pltpu.sync_copy(
                rows1_v.at[pl.ds(0, K)], sacc.at[didx_v.at[j0 + 1]], add=True)
            return carry

        lax.fori_loop(0, PH // 2, body, 0)
    plsc.subcore_barrier()
    pltpu.sync_copy(sacc.at[pl.ds(s * NPT, NPT)],
                    acc_out.at[c].at[pl.ds(s * NPT, NPT)])


@jax.jit
def _sc_scatter(tab, src2d, dst2d):
    return pl.kernel(
        _sc_scatter_body,
        out_type=jax.ShapeDtypeStruct((NC, NPAD, H), _F32),
        mesh=_MESH,
        scratch_types=[
            pltpu.VMEM((PH, K), jnp.int32),
            pltpu.VMEM((PH, K), jnp.int32),
            pltpu.VMEM((KR, H), _F32),
            pltpu.VMEM((KR, H), _F32),
            pltpu.SemaphoreType.DMA,
            pltpu.SemaphoreType.DMA,
            pltpu.VMEM_SHARED((NPAD, H), _F32),
        ],
    )(tab, src2d, dst2d)


DW = 8


def _scale_body(x_ref, wp_ref, bp_ref, w1_ref, cnt_ref, tab_ref, dinv_ref):
    h = jnp.dot(x_ref[...], wp_ref[...], preferred_element_type=_F32)
    h = h + bp_ref[...]
    xw = jnp.dot(h, w1_ref[...], preferred_element_type=_F32)
    deg = cnt_ref[:, 0:1] + cnt_ref[:, 1:2] + 1.0
    dinv = lax.rsqrt(deg)
    tab_ref[...] = dinv * xw
    dinv_ref[...] = jnp.broadcast_to(dinv, (BLK, DW))


@jax.jit
def _tc_scale(x, W_pre, b_pre, W1, cnt):
    return pl.pallas_call(
        _scale_body,
        grid=(N // BLK,),
        in_specs=[
            pl.BlockSpec((BLK, H), lambda i: (i, 0)),
            pl.BlockSpec((H, H), lambda i: (0, 0)),
            pl.BlockSpec((1, H), lambda i: (0, 0)),
            pl.BlockSpec((H, H), lambda i: (0, 0)),
            pl.BlockSpec((BLK, NC), lambda i: (i, 0)),
        ],
        out_specs=[
            pl.BlockSpec((BLK, H), lambda i: (i, 0)),
            pl.BlockSpec((BLK, DW), lambda i: (i, 0)),
        ],
        out_shape=[
            jax.ShapeDtypeStruct((N, H), _F32),
            jax.ShapeDtypeStruct((N, DW), _F32),
        ],
    )(x, W_pre, b_pre, W1, cnt)


def _layer_body(acc_ref, tab_ref, dinv_ref, w_ref, b_ref, out_ref):
    dinv = dinv_ref[:, 0:1]
    agg = acc_ref[0] + acc_ref[1] + tab_ref[...]
    h = jnp.maximum(dinv * agg + b_ref[...], 0.0)
    xw = jnp.dot(h, w_ref[...], preferred_element_type=_F32)
    out_ref[...] = dinv * xw


@jax.jit
def _tc_layer(acc, tab, dinv8, W, b):
    return pl.pallas_call(
        _layer_body,
        grid=(N // BLK,),
        in_specs=[
            pl.BlockSpec((NC, BLK, H), lambda i: (0, i, 0)),
            pl.BlockSpec((BLK, H), lambda i: (i, 0)),
            pl.BlockSpec((BLK, DW), lambda i: (i, 0)),
            pl.BlockSpec((H, H), lambda i: (0, 0)),
            pl.BlockSpec((1, H), lambda i: (0, 0)),
        ],
        out_specs=pl.BlockSpec((BLK, H), lambda i: (i, 0)),
        out_shape=jax.ShapeDtypeStruct((N, H), _F32),
    )(acc, tab, dinv8, W, b)


def _final_body(acc_ref, tab_ref, dinv_ref, b2_ref, wpost_ref, bpost_ref, out_ref):
    dinv = dinv_ref[:, 0:1]
    agg = acc_ref[0] + acc_ref[1] + tab_ref[...]
    h = jnp.maximum(dinv * agg + b2_ref[...], 0.0)
    out_ref[...] = jnp.dot(h, wpost_ref[...],
                           preferred_element_type=_F32) + bpost_ref[...]


@jax.jit
def _tc_final(acc, tab, dinv8, b2, W_post, b_post):
    return pl.pallas_call(
        _final_body,
        grid=(N // BLK,),
        in_specs=[
            pl.BlockSpec((NC, BLK, H), lambda i: (0, i, 0)),
            pl.BlockSpec((BLK, H), lambda i: (i, 0)),
            pl.BlockSpec((BLK, DW), lambda i: (i, 0)),
            pl.BlockSpec((1, H), lambda i: (0, 0)),
            pl.BlockSpec((H, C), lambda i: (0, 0)),
            pl.BlockSpec((1, C), lambda i: (0, 0)),
        ],
        out_specs=pl.BlockSpec((BLK, C), lambda i: (i, 0)),
        out_shape=jax.ShapeDtypeStruct((N, C), _F32),
    )(acc, tab, dinv8, b2, W_post, b_post)


def kernel(x, edge_index, W_pre, b_pre, W1, b1, W2, b2, W_post, b_post):
    src2d = edge_index[0].reshape(EB, K)
    dst2d = edge_index[1].reshape(EB, K)
    ones_c = jnp.ones((K,), _F32)
    zeros_c = jnp.zeros((NPT,), _F32)

    cnt = _sc_count(dst2d, ones_c, zeros_c)
    tab1, dinv8 = _tc_scale(x, W_pre, b_pre.reshape(1, H), W1, cnt.T)
    acc1 = _sc_scatter(tab1, src2d, dst2d)
    tab2 = _tc_layer(acc1, tab1, dinv8, W2, b1.reshape(1, H))
    acc2 = _sc_scatter(tab2, src2d, dst2d)
    return _tc_final(acc2, tab2, dinv8, b2.reshape(1, H),
                     W_post, b_post.reshape(1, C))
